# Initial kernel scaffold; baseline (speedup 1.0000x reference)
import jax
import jax.numpy as jnp
from jax.experimental import pallas as pl


def kernel(probas, labels, matchings):
    raise NotImplementedError("write your pallas kernel here")


# speedup vs baseline: 36.5838x; 36.5838x over previous
import jax
import jax.numpy as jnp
from jax.experimental import pallas as pl
from jax.experimental.pallas import tpu as pltpu

_N_BINS = 10
_BLOCK = 2000


def _body(edges_ref, probas_ref, labels_ref, match_ref, out_ref, acc_ref):
    i = pl.program_id(0)
    nblk = pl.num_programs(0)

    @pl.when(i == 0)
    def _init():
        acc_ref[...] = jnp.zeros_like(acc_ref)

    pb = probas_ref[...]
    lab = labels_ref[...]
    m = match_ref[...]
    edges = edges_ref[...]

    ncol = pb.shape[1]
    cls = jax.lax.broadcasted_iota(jnp.int32, (1, ncol), 1)
    L = jnp.where(lab == cls, m, 0.0)

    cnt_rows, sp_rows, tp_rows = [], [], []
    for j in range(_N_BINS):
        e = edges[0:1, j:j + 1]
        cmp = pb > e
        cnt_rows.append(jnp.sum(jnp.where(cmp, 1.0, 0.0), axis=0, keepdims=True))
        sp_rows.append(jnp.sum(jnp.where(cmp, pb, 0.0), axis=0, keepdims=True))
        tp_rows.append(jnp.sum(jnp.where(cmp, L, 0.0), axis=0, keepdims=True))

    acc_ref[0:_N_BINS, 0:ncol] += jnp.concatenate(cnt_rows, axis=0)
    acc_ref[16:16 + _N_BINS, 0:ncol] += jnp.concatenate(sp_rows, axis=0)
    acc_ref[32:32 + _N_BINS, 0:ncol] += jnp.concatenate(tp_rows, axis=0)

    @pl.when(i == nblk - 1)
    def _epilogue():
        cnt = acc_ref[0:_N_BINS, 0:ncol]
        sp = acc_ref[16:16 + _N_BINS, 0:ncol]
        tpa = acc_ref[32:32 + _N_BINS, 0:ncol]
        zrow = jnp.zeros((1, ncol), jnp.float32)
        ns = cnt - jnp.concatenate([cnt[1:, :], zrow], axis=0)
        spb = sp - jnp.concatenate([sp[1:, :], zrow], axis=0)
        tpb = tpa - jnp.concatenate([tpa[1:, :], zrow], axis=0)

        total = jnp.sum(ns, axis=0, keepdims=True)
        nonempty = ns > 0.0
        mp = spb / jnp.maximum(ns, 1.0)
        pr = tpb / jnp.maximum(ns, 1e-12)
        pbw = ns / jnp.maximum(total, 1.0)
        term = jnp.where(nonempty, pbw * jnp.square(mp - pr), 0.0)
        s_c = jnp.sum(term, axis=0, keepdims=True)
        sq = jnp.square(jnp.sqrt(s_c))
        lane = jax.lax.broadcasted_iota(jnp.int32, (1, ncol), 1)
        sq = jnp.where(lane < ncol - 1, sq, 0.0)
        out_ref[...] = jnp.sqrt(jnp.sum(sq, axis=1, keepdims=True) / (ncol - 1))


def kernel(probas, labels, matchings):
    n, ncol = probas.shape
    labels2 = labels.reshape(n, 1)
    match2 = matchings.astype(jnp.float32).reshape(n, 1)
    edges = jnp.zeros((1, 16), jnp.float32).at[0, :11].set(
        jnp.linspace(0.0, 1.0, _N_BINS + 1, dtype=jnp.float32))

    grid = n // _BLOCK
    out = pl.pallas_call(
        _body,
        grid=(grid,),
        in_specs=[
            pl.BlockSpec((1, 16), lambda i: (0, 0)),
            pl.BlockSpec((_BLOCK, ncol), lambda i: (i, 0)),
            pl.BlockSpec((_BLOCK, 1), lambda i: (i, 0)),
            pl.BlockSpec((_BLOCK, 1), lambda i: (i, 0)),
        ],
        out_specs=pl.BlockSpec((1, 1), lambda i: (0, 0)),
        out_shape=jax.ShapeDtypeStruct((1, 1), jnp.float32),
        scratch_shapes=[pltpu.VMEM((48, 128), jnp.float32)],
        compiler_params=pltpu.CompilerParams(
            dimension_semantics=("arbitrary",)),
    )(edges, probas, labels2, match2)
    return out[0, 0]


# speedup vs baseline: 96.6100x; 2.6408x over previous
import jax
import jax.numpy as jnp
from jax import lax
from jax.experimental import pallas as pl
from jax.experimental.pallas import tpu as pltpu
from jax.experimental.pallas import tpu_sc as plsc

_N_BINS = 10
_NCOL = 21

_SC_CORES = 2
_SC_SUBCORES = 16
_SC_LANES = 16
_SC_WORKERS = _SC_CORES * _SC_SUBCORES
_SC_CHUNK = 2000
_TBL_PAD = 224

_BR = 2000


def _sc_body(edges_hbm, probas_hbm, labels_hbm, match_hbm, out_hbm,
             rows_v, lab_v, m_v, edges_v, table_v, fold_v):
    wid = lax.axis_index("s") * _SC_CORES + lax.axis_index("c")
    nchunks = probas_hbm.shape[0] // (_SC_CHUNK * _NCOL)

    zz = jnp.zeros((16,), jnp.float32)
    for g in range(_SC_LANES * _TBL_PAD // 16):
        table_v[pl.ds(g * 16, 16)] = zz

    pltpu.sync_copy(edges_hbm, edges_v)
    ev = [edges_v[j, :] for j in range(_N_BINS)]
    lane = lax.iota(jnp.int32, 16)

    n_outer = (nchunks + _SC_WORKERS - 1) // _SC_WORKERS
    for t in range(n_outer):
        chunk = wid + t * _SC_WORKERS

        @pl.when(chunk < nchunks)
        def _do():
            base = chunk * _SC_CHUNK
            pltpu.sync_copy(probas_hbm.at[pl.ds(base * _NCOL,
                                                _SC_CHUNK * _NCOL)], rows_v)
            pltpu.sync_copy(labels_hbm.at[pl.ds(base, _SC_CHUNK)], lab_v)
            pltpu.sync_copy(match_hbm.at[pl.ds(base, _SC_CHUNK)], m_v)

            def step(g, carry):
                off = g * 16
                lab16 = lab_v[pl.ds(off, 16)]
                m16 = m_v[pl.ds(off, 16)]
                flat16 = (lane + off) * _NCOL + lab16
                q16 = plsc.load_gather(rows_v, [flat16])
                s = jnp.zeros((16,), jnp.int32)
                for j in range(_N_BINS):
                    s = s + jnp.where(q16 > ev[j], 1, 0)
                valid = s >= 1
                buck = jnp.where(valid, (s - 1) * _NCOL + lab16, 0)
                val = jnp.where(valid, m16, 0.0)
                plsc.addupdate_scatter(table_v, [lane * _TBL_PAD + buck], val)
                return carry

            lax.fori_loop(0, _SC_CHUNK // 16, step, 0)

    for g in range(_TBL_PAD // 16):
        acc = table_v[pl.ds(g * 16, 16)]
        for l in range(1, _SC_LANES):
            acc = acc + table_v[pl.ds(l * _TBL_PAD + g * 16, 16)]
        fold_v[pl.ds(g * 16, 16)] = acc

    pltpu.sync_copy(fold_v, out_hbm.at[wid])


def _sc_tp(probas, labels, match_f, edges_b):
    mesh = plsc.VectorSubcoreMesh(core_axis_name="c", subcore_axis_name="s")
    fn = pl.kernel(
        _sc_body,
        out_type=jax.ShapeDtypeStruct((_SC_WORKERS, _TBL_PAD), jnp.float32),
        mesh=mesh,
        scratch_types=[
            pltpu.VMEM((_SC_CHUNK * _NCOL,), jnp.float32),
            pltpu.VMEM((_SC_CHUNK,), jnp.int32),
            pltpu.VMEM((_SC_CHUNK,), jnp.float32),
            pltpu.VMEM((_N_BINS, 16), jnp.float32),
            pltpu.VMEM((_SC_LANES * _TBL_PAD,), jnp.float32),
            pltpu.VMEM((_TBL_PAD,), jnp.float32),
        ],
        compiler_params=pltpu.CompilerParams(needs_layout_passes=False),
    )
    return fn(edges_b, probas.reshape(-1), labels, match_f)


def _dense_body(edges_ref, pb_ref, out_ref, acc_ref):
    i = pl.program_id(0)
    n = pl.num_programs(0)

    @pl.when(i == 0)
    def _init():
        acc_ref[...] = jnp.zeros_like(acc_ref)

    pb = pb_ref[...]
    edges = edges_ref[...]
    ones = jnp.ones((1, _BR), jnp.float32)
    dn = (((1,), (0,)), ((), ()))
    rows = []
    for j in range(_N_BINS):
        e = edges[0:1, j:j + 1]
        m = (pb > e).astype(jnp.float32)
        rows.append(lax.dot_general(ones, m, dn,
                                    preferred_element_type=jnp.float32))
    for j in range(_N_BINS):
        e = edges[0:1, j:j + 1]
        pm = jnp.where(pb > e, pb, 0.0)
        rows.append(lax.dot_general(ones, pm, dn,
                                    preferred_element_type=jnp.float32))
    acc_ref[0:2 * _N_BINS, 0:105] += jnp.concatenate(rows, axis=0)

    @pl.when(i == n - 1)
    def _fin():
        a = acc_ref[...]
        folded = (a[:, 0:21] + a[:, 21:42] + a[:, 42:63] + a[:, 63:84]
                  + a[:, 84:105])
        out_ref[...] = jnp.concatenate(
            [folded, jnp.zeros((32, 128 - _NCOL), jnp.float32)], axis=1)


def _combine_body(stats_ref, tp_ref, out_ref):
    a = stats_ref[...]
    tp3 = tp_ref[...]
    tpb = jnp.sum(tp3, axis=0)
    cnt = a[0:_N_BINS, 0:_NCOL]
    sp = a[_N_BINS:2 * _N_BINS, 0:_NCOL]
    z = jnp.zeros((1, _NCOL), jnp.float32)
    ns = cnt - jnp.concatenate([cnt[1:, :], z], axis=0)
    spb = sp - jnp.concatenate([sp[1:, :], z], axis=0)
    total = jnp.sum(ns, axis=0, keepdims=True)
    mp = spb / jnp.maximum(ns, 1.0)
    pr = tpb / jnp.maximum(ns, 1e-12)
    pbw = ns / jnp.maximum(total, 1.0)
    term = jnp.where(ns > 0.0, pbw * jnp.square(mp - pr), 0.0)
    s_c = jnp.sum(term, axis=0, keepdims=True)
    sq = jnp.square(jnp.sqrt(s_c))
    lidx = lax.broadcasted_iota(jnp.int32, (1, _NCOL), 1)
    sq = jnp.where(lidx < _NCOL - 1, sq, 0.0)
    out_ref[...] = jnp.sqrt(jnp.sum(sq, axis=1, keepdims=True) / (_NCOL - 1))


def kernel(probas, labels, matchings):
    n, ncol = probas.shape
    edges_full = jnp.linspace(0.0, 1.0, _N_BINS + 1, dtype=jnp.float32)
    edges16 = jnp.zeros((1, 16), jnp.float32).at[0, :11].set(edges_full)
    edges_b = jnp.broadcast_to(edges_full[:_N_BINS, None], (_N_BINS, 16))
    match_f = matchings.astype(jnp.float32)

    tp_part = _sc_tp(probas, labels, match_f, edges_b)
    tp3 = tp_part[:, :_N_BINS * _NCOL].reshape(_SC_WORKERS, _N_BINS, _NCOL)

    pb105 = probas.reshape(n // 5, 5 * ncol)
    stats = pl.pallas_call(
        _dense_body,
        grid=(n // 5 // _BR,),
        in_specs=[
            pl.BlockSpec((1, 16), lambda i: (0, 0)),
            pl.BlockSpec((_BR, 5 * _NCOL), lambda i: (i, 0)),
        ],
        out_specs=pl.BlockSpec((32, 128), lambda i: (0, 0)),
        out_shape=jax.ShapeDtypeStruct((32, 128), jnp.float32),
        scratch_shapes=[pltpu.VMEM((32, 128), jnp.float32)],
        compiler_params=pltpu.CompilerParams(
            dimension_semantics=("arbitrary",)),
    )(edges16, pb105)

    out = pl.pallas_call(
        _combine_body,
        grid=(1,),
        in_specs=[
            pl.BlockSpec((32, 128), lambda i: (0, 0)),
            pl.BlockSpec((_SC_WORKERS, _N_BINS, _NCOL), lambda i: (0, 0, 0)),
        ],
        out_specs=pl.BlockSpec((1, 1), lambda i: (0, 0)),
        out_shape=jax.ShapeDtypeStruct((1, 1), jnp.float32),
    )(stats, tp3)
    return out[0, 0]


# speedup vs baseline: 164.0550x; 1.6981x over previous
import jax
import jax.numpy as jnp
from jax import lax
from jax.experimental import pallas as pl
from jax.experimental.pallas import tpu as pltpu
from jax.experimental.pallas import tpu_sc as plsc

_N_BINS = 10
_NCOL = 21

_SC_CORES = 2
_SC_SUBCORES = 16
_SC_LANES = 16
_SC_WORKERS = _SC_CORES * _SC_SUBCORES
_SC_CHUNK = 400
_TBL_PAD = 224

_BR = 4000
_GRP = 5


def _sc_body(edges_hbm, probas_hbm, labels_hbm, match_hbm, out_hbm,
             rows_v, lab_v, m_v, edges_v, table_v, fold_v):
    wid = lax.axis_index("s") * _SC_CORES + lax.axis_index("c")
    nchunks = probas_hbm.shape[0] // _SC_CHUNK

    zz = jnp.zeros((16,), jnp.float32)
    for g in range(_SC_LANES * _TBL_PAD // 16):
        table_v[pl.ds(g * 16, 16)] = zz

    pltpu.sync_copy(edges_hbm, edges_v)
    ev = [edges_v[j, :] for j in range(_N_BINS)]
    lane = lax.iota(jnp.int32, 16)

    n_outer = (nchunks + _SC_WORKERS - 1) // _SC_WORKERS
    for t in range(n_outer):
        chunk = wid + t * _SC_WORKERS

        @pl.when(chunk < nchunks)
        def _do():
            base = chunk * _SC_CHUNK
            pltpu.sync_copy(probas_hbm.at[pl.ds(base, _SC_CHUNK)], rows_v)
            pltpu.sync_copy(labels_hbm.at[pl.ds(base, _SC_CHUNK)], lab_v)
            pltpu.sync_copy(match_hbm.at[pl.ds(base, _SC_CHUNK)], m_v)

            def step(g, carry):
                off = g * 16
                lab16 = lab_v[pl.ds(off, 16)]
                m16 = m_v[pl.ds(off, 16)]
                row16 = lane + off
                q16 = plsc.load_gather(rows_v, [row16, lab16])
                s = jnp.zeros((16,), jnp.int32)
                for j in range(_N_BINS):
                    s = s + jnp.where(q16 > ev[j], 1, 0)
                valid = s >= 1
                buck = jnp.where(valid, (s - 1) * _NCOL + lab16, 0)
                val = jnp.where(valid, m16, 0.0)
                plsc.addupdate_scatter(table_v, [lane * _TBL_PAD + buck], val)
                return carry

            lax.fori_loop(0, _SC_CHUNK // 16, step, 0)

    for g in range(_TBL_PAD // 16):
        acc = table_v[pl.ds(g * 16, 16)]
        for l in range(1, _SC_LANES):
            acc = acc + table_v[pl.ds(l * _TBL_PAD + g * 16, 16)]
        fold_v[pl.ds(g * 16, 16)] = acc

    pltpu.sync_copy(fold_v, out_hbm.at[wid])


def _sc_tp(probas, labels, match_f, edges_b):
    mesh = plsc.VectorSubcoreMesh(core_axis_name="c", subcore_axis_name="s")
    fn = pl.kernel(
        _sc_body,
        out_type=jax.ShapeDtypeStruct((_SC_WORKERS, _TBL_PAD), jnp.float32),
        mesh=mesh,
        scratch_types=[
            pltpu.VMEM((_SC_CHUNK, _NCOL), jnp.float32),
            pltpu.VMEM((_SC_CHUNK,), jnp.int32),
            pltpu.VMEM((_SC_CHUNK,), jnp.float32),
            pltpu.VMEM((_N_BINS, 16), jnp.float32),
            pltpu.VMEM((_SC_LANES * _TBL_PAD,), jnp.float32),
            pltpu.VMEM((_TBL_PAD,), jnp.float32),
        ],
        compiler_params=pltpu.CompilerParams(needs_layout_passes=False),
    )
    return fn(edges_b, probas, labels, match_f)


def _dense_body(edges_ref, pb_ref, out_ref, acc_ref):
    i = pl.program_id(0)
    n = pl.num_programs(0)

    @pl.when(i == 0)
    def _init():
        acc_ref[...] = jnp.zeros_like(acc_ref)

    pb21 = pb_ref[...]
    sub = _BR // _GRP
    pb = jnp.concatenate(
        [pb21[k * sub:(k + 1) * sub, :] for k in range(_GRP)], axis=1)
    edges = edges_ref[...]
    ones = jnp.ones((1, sub), jnp.float32)
    dn = (((1,), (0,)), ((), ()))
    rows = []
    for j in range(_N_BINS):
        e = edges[0:1, j:j + 1]
        m = (pb > e).astype(jnp.float32)
        rows.append(lax.dot_general(ones, m, dn,
                                    preferred_element_type=jnp.float32))
    for j in range(_N_BINS):
        e = edges[0:1, j:j + 1]
        pm = jnp.where(pb > e, pb, 0.0)
        rows.append(lax.dot_general(ones, pm, dn,
                                    preferred_element_type=jnp.float32))
    acc_ref[0:2 * _N_BINS, 0:_GRP * _NCOL] += jnp.concatenate(rows, axis=0)

    @pl.when(i == n - 1)
    def _fin():
        a = acc_ref[...]
        folded = (a[:, 0:21] + a[:, 21:42] + a[:, 42:63] + a[:, 63:84]
                  + a[:, 84:105])
        out_ref[...] = jnp.concatenate(
            [folded, jnp.zeros((32, 128 - _NCOL), jnp.float32)], axis=1)


def _combine_body(stats_ref, tp_ref, out_ref):
    a = stats_ref[...]
    tp3 = tp_ref[...]
    tpb = jnp.sum(tp3, axis=0)
    cnt = a[0:_N_BINS, 0:_NCOL]
    sp = a[_N_BINS:2 * _N_BINS, 0:_NCOL]
    z = jnp.zeros((1, _NCOL), jnp.float32)
    ns = cnt - jnp.concatenate([cnt[1:, :], z], axis=0)
    spb = sp - jnp.concatenate([sp[1:, :], z], axis=0)
    total = jnp.sum(ns, axis=0, keepdims=True)
    mp = spb / jnp.maximum(ns, 1.0)
    pr = tpb / jnp.maximum(ns, 1e-12)
    pbw = ns / jnp.maximum(total, 1.0)
    term = jnp.where(ns > 0.0, pbw * jnp.square(mp - pr), 0.0)
    s_c = jnp.sum(term, axis=0, keepdims=True)
    sq = jnp.square(jnp.sqrt(s_c))
    lidx = lax.broadcasted_iota(jnp.int32, (1, _NCOL), 1)
    sq = jnp.where(lidx < _NCOL - 1, sq, 0.0)
    out_ref[...] = jnp.sqrt(jnp.sum(sq, axis=1, keepdims=True) / (_NCOL - 1))


def kernel(probas, labels, matchings):
    n, ncol = probas.shape
    edges_full = jnp.linspace(0.0, 1.0, _N_BINS + 1, dtype=jnp.float32)
    edges16 = jnp.zeros((1, 16), jnp.float32).at[0, :11].set(edges_full)
    edges_b = jnp.broadcast_to(edges_full[:_N_BINS, None], (_N_BINS, 16))
    match_f = matchings.astype(jnp.float32)

    tp_part = _sc_tp(probas, labels, match_f, edges_b)
    tp3 = tp_part[:, :_N_BINS * _NCOL].reshape(_SC_WORKERS, _N_BINS, _NCOL)

    stats = pl.pallas_call(
        _dense_body,
        grid=(n // _BR,),
        in_specs=[
            pl.BlockSpec((1, 16), lambda i: (0, 0)),
            pl.BlockSpec((_BR, _NCOL), lambda i: (i, 0)),
        ],
        out_specs=pl.BlockSpec((32, 128), lambda i: (0, 0)),
        out_shape=jax.ShapeDtypeStruct((32, 128), jnp.float32),
        scratch_shapes=[pltpu.VMEM((32, 128), jnp.float32)],
        compiler_params=pltpu.CompilerParams(
            dimension_semantics=("arbitrary",)),
    )(edges16, probas)

    out = pl.pallas_call(
        _combine_body,
        grid=(1,),
        in_specs=[
            pl.BlockSpec((32, 128), lambda i: (0, 0)),
            pl.BlockSpec((_SC_WORKERS, _N_BINS, _NCOL), lambda i: (0, 0, 0)),
        ],
        out_specs=pl.BlockSpec((1, 1), lambda i: (0, 0)),
        out_shape=jax.ShapeDtypeStruct((1, 1), jnp.float32),
    )(stats, tp3)
    return out[0, 0]
